# baseline (speedup 1.0000x reference)
import functools

import jax
import jax.numpy as jnp
from jax import lax
from jax.experimental import pallas as pl
from jax.experimental.pallas import tpu as pltpu
from jax.experimental.pallas import tpu_sc as plsc

BATCH = 16384
INPUT_DIM = 39
N_CONT = 13
N_CAT = 26
EMB_DIM = 16
OUT_DIM = N_CONT + N_CAT * EMB_DIM

_NUM_CORES = 2
_NUM_SUBCORES = 16
NW = _NUM_CORES * _NUM_SUBCORES
BPW = BATCH // NW
CHUNK = 128
N_CHUNKS = BPW // CHUNK
LANES = 16


def _body(x_hbm, *args):
    tables = args[:N_CAT]
    out_hbm = args[N_CAT]
    x_v, idxs_v, stage_v = args[N_CAT + 1:N_CAT + 4]
    ebufs = args[N_CAT + 4:N_CAT + 4 + N_CAT]
    sem = args[N_CAT + 4 + N_CAT]

    wid = lax.axis_index("s") * _NUM_CORES + lax.axis_index("c")
    base = wid * BPW

    iota = lax.iota(jnp.int32, LANES)

    def chunk_body(c, carry):
        cbase = base + c * CHUNK
        pltpu.sync_copy(x_hbm.at[pl.ds(cbase, CHUNK)], x_v)

        def extract(b0, carry2):
            rows = iota + b0 * LANES
            off = pl.multiple_of(b0 * LANES, LANES)
            for j in range(INPUT_DIM):
                col = jnp.full((LANES,), j, jnp.int32)
                v = plsc.load_gather(x_v, [rows, col])
                if j < N_CONT:
                    stage_v[j, pl.ds(off, LANES)] = v.astype(jnp.float32)
                else:
                    plsc.store_scatter(
                        idxs_v, [jnp.full((LANES,), j - N_CONT, jnp.int32),
                                 rows], v)
            return carry2

        lax.fori_loop(0, CHUNK // LANES, extract, 0)

        for t in range(N_CAT):
            pltpu.make_async_copy(
                tables[t].at[idxs_v.at[t]], ebufs[t], sem).start()

        for t in range(N_CAT):
            pltpu.make_async_copy(
                tables[t].at[idxs_v.at[t]], ebufs[t], sem).wait()

            def asm_body(b0, carry2, t=t):
                brows = iota + b0 * LANES
                off = pl.multiple_of(b0 * LANES, LANES)
                for e in range(EMB_DIM):
                    v = plsc.load_gather(
                        ebufs[t], [brows, jnp.full((LANES,), e, jnp.int32)])
                    stage_v[N_CONT + t * EMB_DIM + e, pl.ds(off, LANES)] = v
                return carry2

            lax.fori_loop(0, CHUNK // LANES, asm_body, 0)

        pltpu.sync_copy(stage_v, out_hbm.at[:, pl.ds(cbase, CHUNK)])
        return carry

    lax.fori_loop(0, N_CHUNKS, chunk_body, 0)


_emb_kernel = functools.partial(
    pl.kernel,
    out_type=jax.ShapeDtypeStruct((OUT_DIM, BATCH), jnp.float32),
    mesh=plsc.VectorSubcoreMesh(core_axis_name="c", subcore_axis_name="s"),
    scratch_types=[
        pltpu.VMEM((CHUNK, INPUT_DIM), jnp.int32),
        pltpu.VMEM((N_CAT, CHUNK), jnp.int32),
        pltpu.VMEM((OUT_DIM, CHUNK), jnp.float32),
    ] + [pltpu.VMEM((CHUNK, EMB_DIM), jnp.float32) for _ in range(N_CAT)] + [
        pltpu.SemaphoreType.DMA,
    ],
    compiler_params=pltpu.CompilerParams(use_tc_tiling_on_sc=False,
                                         needs_layout_passes=False),
)(_body)


def kernel(x, emb_0, emb_1, emb_2, emb_3, emb_4, emb_5, emb_6, emb_7,
           emb_8, emb_9, emb_10, emb_11, emb_12, emb_13, emb_14, emb_15,
           emb_16, emb_17, emb_18, emb_19, emb_20, emb_21, emb_22, emb_23,
           emb_24, emb_25):
    tables = (emb_0, emb_1, emb_2, emb_3, emb_4, emb_5, emb_6, emb_7,
              emb_8, emb_9, emb_10, emb_11, emb_12, emb_13, emb_14, emb_15,
              emb_16, emb_17, emb_18, emb_19, emb_20, emb_21, emb_22, emb_23,
              emb_24, emb_25)
    out_t = _emb_kernel(x.astype(jnp.int32), *tables)
    return out_t.T


# speedup vs baseline: 1.0122x; 1.0122x over previous
import functools

import jax
import jax.numpy as jnp
from jax import lax
from jax.experimental import pallas as pl
from jax.experimental.pallas import tpu as pltpu
from jax.experimental.pallas import tpu_sc as plsc

BATCH = 16384
INPUT_DIM = 39
N_CONT = 13
N_CAT = 26
EMB_DIM = 16
OUT_DIM = N_CONT + N_CAT * EMB_DIM

_NUM_CORES = 2
_NUM_SUBCORES = 16
NW = _NUM_CORES * _NUM_SUBCORES
BPW = BATCH // NW
CHUNK = 128
N_CHUNKS = BPW // CHUNK
LANES = 16

_MESH = plsc.VectorSubcoreMesh(core_axis_name="c", subcore_axis_name="s")
_CPARAMS = pltpu.CompilerParams(use_tc_tiling_on_sc=False,
                                needs_layout_passes=False)


def _wid():
    return lax.axis_index("s") * _NUM_CORES + lax.axis_index("c")


def _extract_body(x_hbm, idxs_hbm, cont_hbm, x_v, i_stage, c_stage):
    base = _wid() * BPW
    iota = lax.iota(jnp.int32, LANES)
    pltpu.sync_copy(x_hbm.at[pl.ds(base, BPW)], x_v)

    def extract(b0, carry):
        rows = iota + b0 * LANES
        for j in range(INPUT_DIM):
            col = jnp.full((LANES,), j, jnp.int32)
            v = plsc.load_gather(x_v, [rows, col])
            if j < N_CONT:
                plsc.store_scatter(
                    c_stage, [jnp.full((LANES,), j, jnp.int32), rows],
                    v.astype(jnp.float32))
            else:
                plsc.store_scatter(
                    i_stage, [jnp.full((LANES,), j - N_CONT, jnp.int32),
                              rows], v)
        return carry

    lax.fori_loop(0, BPW // LANES, extract, 0)
    pltpu.sync_copy(i_stage, idxs_hbm.at[:, pl.ds(base, BPW)])
    pltpu.sync_copy(c_stage, cont_hbm.at[:, pl.ds(base, BPW)])


_extract_kernel = functools.partial(
    pl.kernel,
    out_type=(jax.ShapeDtypeStruct((N_CAT, BATCH), jnp.int32),
              jax.ShapeDtypeStruct((N_CONT, BATCH), jnp.float32)),
    mesh=_MESH,
    scratch_types=[
        pltpu.VMEM((BPW, INPUT_DIM), jnp.int32),
        pltpu.VMEM((N_CAT, BPW), jnp.int32),
        pltpu.VMEM((N_CONT, BPW), jnp.float32),
    ],
    compiler_params=_CPARAMS,
)(_extract_body)


def _gather_body(t, table_hbm, idxs_hbm, piece_hbm, idx_v, ebuf, stage, sem):
    base = _wid() * BPW
    iota = lax.iota(jnp.int32, LANES)
    pltpu.sync_copy(idxs_hbm.at[t, pl.ds(base, BPW)], idx_v)
    gather = pltpu.make_async_copy(table_hbm.at[idx_v], ebuf, sem)
    gather.start()
    gather.wait()

    def asm(b0, carry):
        brows = iota + b0 * LANES
        off = pl.multiple_of(b0 * LANES, LANES)
        for e in range(EMB_DIM):
            v = plsc.load_gather(
                ebuf, [brows, jnp.full((LANES,), e, jnp.int32)])
            stage[e, pl.ds(off, LANES)] = v
        return carry

    lax.fori_loop(0, BPW // LANES, asm, 0)
    pltpu.sync_copy(stage, piece_hbm.at[:, pl.ds(base, BPW)])


def _make_gather_kernel(t):
    return functools.partial(
        pl.kernel,
        out_type=jax.ShapeDtypeStruct((EMB_DIM, BATCH), jnp.float32),
        mesh=_MESH,
        scratch_types=[
            pltpu.VMEM((BPW,), jnp.int32),
            pltpu.VMEM((BPW, EMB_DIM), jnp.float32),
            pltpu.VMEM((EMB_DIM, BPW), jnp.float32),
            pltpu.SemaphoreType.DMA,
        ],
        compiler_params=_CPARAMS,
        name=f"gather_t{t}",
    )(functools.partial(_gather_body, t))


_gather_kernels = [_make_gather_kernel(t) for t in range(N_CAT)]


def _concat_body(*args):
    cont_hbm = args[0]
    pieces = args[1:1 + N_CAT]
    out_hbm = args[1 + N_CAT]
    stage_v = args[2 + N_CAT]
    base = _wid() * BPW

    def chunk_body(c, carry):
        cb = base + c * CHUNK
        pltpu.sync_copy(cont_hbm.at[:, pl.ds(cb, CHUNK)],
                        stage_v.at[pl.ds(0, N_CONT)])
        for t in range(N_CAT):
            pltpu.sync_copy(
                pieces[t].at[:, pl.ds(cb, CHUNK)],
                stage_v.at[pl.ds(N_CONT + t * EMB_DIM, EMB_DIM)])
        pltpu.sync_copy(stage_v, out_hbm.at[:, pl.ds(cb, CHUNK)])
        return carry

    lax.fori_loop(0, N_CHUNKS, chunk_body, 0)


_concat_kernel = functools.partial(
    pl.kernel,
    out_type=jax.ShapeDtypeStruct((OUT_DIM, BATCH), jnp.float32),
    mesh=_MESH,
    scratch_types=[
        pltpu.VMEM((OUT_DIM, CHUNK), jnp.float32),
    ],
    compiler_params=_CPARAMS,
)(_concat_body)


def kernel(x, emb_0, emb_1, emb_2, emb_3, emb_4, emb_5, emb_6, emb_7,
           emb_8, emb_9, emb_10, emb_11, emb_12, emb_13, emb_14, emb_15,
           emb_16, emb_17, emb_18, emb_19, emb_20, emb_21, emb_22, emb_23,
           emb_24, emb_25):
    tables = (emb_0, emb_1, emb_2, emb_3, emb_4, emb_5, emb_6, emb_7,
              emb_8, emb_9, emb_10, emb_11, emb_12, emb_13, emb_14, emb_15,
              emb_16, emb_17, emb_18, emb_19, emb_20, emb_21, emb_22, emb_23,
              emb_24, emb_25)
    idxs, cont = _extract_kernel(x.astype(jnp.int32))
    pieces = [_gather_kernels[t](tables[t], idxs) for t in range(N_CAT)]
    out_t = _concat_kernel(cont, *pieces)
    return out_t.T


# speedup vs baseline: 1.2443x; 1.2293x over previous
import functools

import jax
import jax.numpy as jnp
from jax import lax
from jax.experimental import pallas as pl
from jax.experimental.pallas import tpu as pltpu
from jax.experimental.pallas import tpu_sc as plsc

BATCH = 16384
INPUT_DIM = 39
N_CONT = 13
N_CAT = 26
EMB_DIM = 16
OUT_DIM = N_CONT + N_CAT * EMB_DIM

_NUM_CORES = 2
_NUM_SUBCORES = 16
NW = _NUM_CORES * _NUM_SUBCORES
BPW = BATCH // NW
CHUNK = 128
N_CHUNKS = BPW // CHUNK
LANES = 16

_MESH = plsc.VectorSubcoreMesh(core_axis_name="c", subcore_axis_name="s")
_CPARAMS = pltpu.CompilerParams(use_tc_tiling_on_sc=False,
                                needs_layout_passes=False)


def _wid():
    return lax.axis_index("s") * _NUM_CORES + lax.axis_index("c")


def _extract_body(x_hbm, idxs_hbm, cont_hbm, x_v, i_stage, c_stage):
    base = _wid() * BPW
    iota = lax.iota(jnp.int32, LANES)
    pltpu.sync_copy(x_hbm.at[pl.ds(base, BPW)], x_v)

    def extract(b0, carry):
        rows = iota + b0 * LANES
        for j in range(INPUT_DIM):
            col = jnp.full((LANES,), j, jnp.int32)
            v = plsc.load_gather(x_v, [rows, col])
            if j < N_CONT:
                plsc.store_scatter(
                    c_stage, [jnp.full((LANES,), j, jnp.int32), rows],
                    v.astype(jnp.float32))
            else:
                plsc.store_scatter(
                    i_stage, [jnp.full((LANES,), j - N_CONT, jnp.int32),
                              rows], v)
        return carry

    lax.fori_loop(0, BPW // LANES, extract, 0)
    pltpu.sync_copy(i_stage, idxs_hbm.at[:, pl.ds(base, BPW)])
    pltpu.sync_copy(c_stage, cont_hbm.at[:, pl.ds(base, BPW)])


_extract_kernel = functools.partial(
    pl.kernel,
    out_type=(jax.ShapeDtypeStruct((N_CAT, BATCH), jnp.int32),
              jax.ShapeDtypeStruct((N_CONT, BATCH), jnp.float32)),
    mesh=_MESH,
    scratch_types=[
        pltpu.VMEM((BPW, INPUT_DIM), jnp.int32),
        pltpu.VMEM((N_CAT, BPW), jnp.int32),
        pltpu.VMEM((N_CONT, BPW), jnp.float32),
    ],
    compiler_params=_CPARAMS,
)(_extract_body)


_CPARAMS_TILED = pltpu.CompilerParams(use_tc_tiling_on_sc=True,
                                      needs_layout_passes=False)

_CAT_DIM = 100000
_COLS_PW = 3200
_TAIL_W = NW - 1
_TAIL_MAIN = 768
_TAIL_PART = 32
_TAIL_COLS = _TAIL_MAIN + _TAIL_PART
_CONV_SPLIT = N_CAT // 2


def _convert_body(tab_lo, tab_hi, *args):
    tabs = args[:tab_hi - tab_lo]
    outs = args[tab_hi - tab_lo:2 * (tab_hi - tab_lo)]
    slab_v, tail_v, stage_v = args[2 * (tab_hi - tab_lo):]

    wid = _wid()
    iota = lax.iota(jnp.int32, LANES)
    col0 = wid * _COLS_PW
    is_tail = wid == _TAIL_W
    n_iters = jnp.where(is_tail, _TAIL_MAIN // LANES, _COLS_PW // LANES)
    n_out = jnp.where(is_tail, _TAIL_COLS * EMB_DIM, _COLS_PW * EMB_DIM)

    for i in range(tab_hi - tab_lo):
        @pl.when(jnp.logical_not(is_tail))
        def _load_main(i=i):
            pltpu.sync_copy(tabs[i].at[:, pl.ds(col0, _COLS_PW)], slab_v)

        @pl.when(is_tail)
        def _load_tail(i=i):
            pltpu.sync_copy(tabs[i].at[:, pl.ds(col0, _TAIL_MAIN)],
                            slab_v.at[:, pl.ds(0, _TAIL_MAIN)])
            pltpu.sync_copy(
                tabs[i].at[:, pl.ds(_CAT_DIM - _TAIL_PART, _TAIL_PART)],
                tail_v)

        def transpose(b0, carry):
            rcols = iota + b0 * LANES
            for e in range(EMB_DIM):
                v = plsc.load_gather(
                    slab_v, [jnp.full((LANES,), e, jnp.int32), rcols])
                plsc.store_scatter(stage_v, [rcols * EMB_DIM + e], v)
            return carry

        lax.fori_loop(0, n_iters, transpose, 0)

        @pl.when(is_tail)
        def _transpose_tail(i=i):
            def ttail(b0, carry):
                rloc = iota + b0 * LANES
                for e in range(EMB_DIM):
                    v = plsc.load_gather(
                        tail_v, [jnp.full((LANES,), e, jnp.int32), rloc])
                    plsc.store_scatter(
                        stage_v, [(rloc + _TAIL_MAIN) * EMB_DIM + e], v)
                return carry

            lax.fori_loop(0, _TAIL_PART // LANES, ttail, 0)

        @pl.when(jnp.logical_not(is_tail))
        def _store_main(i=i):
            pltpu.sync_copy(
                stage_v.at[pl.ds(0, _COLS_PW * EMB_DIM)],
                outs[i].at[pl.ds(col0 * EMB_DIM, _COLS_PW * EMB_DIM)])

        @pl.when(is_tail)
        def _store_tail(i=i):
            pltpu.sync_copy(
                stage_v.at[pl.ds(0, _TAIL_COLS * EMB_DIM)],
                outs[i].at[pl.ds(col0 * EMB_DIM, _TAIL_COLS * EMB_DIM)])


def _make_convert_kernel(lo, hi):
    return functools.partial(
        pl.kernel,
        out_type=tuple(
            jax.ShapeDtypeStruct((_CAT_DIM * EMB_DIM,), jnp.float32)
            for _ in range(hi - lo)),
        mesh=_MESH,
        scratch_types=[
            pltpu.VMEM((EMB_DIM, _COLS_PW), jnp.float32),
            pltpu.VMEM((EMB_DIM, _TAIL_PART), jnp.float32),
            pltpu.VMEM((_COLS_PW * EMB_DIM,), jnp.float32),
        ],
        compiler_params=_CPARAMS_TILED,
        name=f"convert_{lo}_{hi}",
    )(functools.partial(_convert_body, lo, hi))


_convert_kernels = (_make_convert_kernel(0, _CONV_SPLIT),
                    _make_convert_kernel(_CONV_SPLIT, N_CAT))


def _gather_body(t, table_hbm, idxs_hbm, piece_hbm, idx_v, ebuf, stage, sem):
    base = _wid() * BPW
    iota = lax.iota(jnp.int32, LANES)
    pltpu.sync_copy(idxs_hbm.at[t, pl.ds(base, BPW)], idx_v)
    gather = pltpu.make_async_copy(table_hbm.at[idx_v], ebuf, sem)
    gather.start()
    gather.wait()

    def asm(b0, carry):
        brows = iota + b0 * LANES
        off = pl.multiple_of(b0 * LANES, LANES)
        for e in range(EMB_DIM):
            v = plsc.load_gather(
                ebuf, [brows, jnp.full((LANES,), e, jnp.int32)])
            stage[e, pl.ds(off, LANES)] = v
        return carry

    lax.fori_loop(0, BPW // LANES, asm, 0)
    pltpu.sync_copy(stage, piece_hbm.at[:, pl.ds(base, BPW)])


def _make_gather_kernel(t):
    return functools.partial(
        pl.kernel,
        out_type=jax.ShapeDtypeStruct((EMB_DIM, BATCH), jnp.float32),
        mesh=_MESH,
        scratch_types=[
            pltpu.VMEM((BPW,), jnp.int32),
            pltpu.VMEM((BPW, EMB_DIM), jnp.float32),
            pltpu.VMEM((EMB_DIM, BPW), jnp.float32),
            pltpu.SemaphoreType.DMA,
        ],
        compiler_params=_CPARAMS,
        name=f"gather_t{t}",
    )(functools.partial(_gather_body, t))


_gather_kernels = [_make_gather_kernel(t) for t in range(N_CAT)]


def _concat_body(*args):
    cont_hbm = args[0]
    pieces = args[1:1 + N_CAT]
    out_hbm = args[1 + N_CAT]
    stage_v = args[2 + N_CAT]
    base = _wid() * BPW

    def chunk_body(c, carry):
        cb = base + c * CHUNK
        pltpu.sync_copy(cont_hbm.at[:, pl.ds(cb, CHUNK)],
                        stage_v.at[pl.ds(0, N_CONT)])
        for t in range(N_CAT):
            pltpu.sync_copy(
                pieces[t].at[:, pl.ds(cb, CHUNK)],
                stage_v.at[pl.ds(N_CONT + t * EMB_DIM, EMB_DIM)])
        pltpu.sync_copy(stage_v, out_hbm.at[:, pl.ds(cb, CHUNK)])
        return carry

    lax.fori_loop(0, N_CHUNKS, chunk_body, 0)


_concat_kernel = functools.partial(
    pl.kernel,
    out_type=jax.ShapeDtypeStruct((OUT_DIM, BATCH), jnp.float32),
    mesh=_MESH,
    scratch_types=[
        pltpu.VMEM((OUT_DIM, CHUNK), jnp.float32),
    ],
    compiler_params=_CPARAMS,
)(_concat_body)


def kernel(x, emb_0, emb_1, emb_2, emb_3, emb_4, emb_5, emb_6, emb_7,
           emb_8, emb_9, emb_10, emb_11, emb_12, emb_13, emb_14, emb_15,
           emb_16, emb_17, emb_18, emb_19, emb_20, emb_21, emb_22, emb_23,
           emb_24, emb_25):
    tables = (emb_0, emb_1, emb_2, emb_3, emb_4, emb_5, emb_6, emb_7,
              emb_8, emb_9, emb_10, emb_11, emb_12, emb_13, emb_14, emb_15,
              emb_16, emb_17, emb_18, emb_19, emb_20, emb_21, emb_22, emb_23,
              emb_24, emb_25)
    idxs, cont = _extract_kernel(x.astype(jnp.int32))
    flats_lo = _convert_kernels[0](
        *(tables[t].T for t in range(_CONV_SPLIT)))
    flats_hi = _convert_kernels[1](
        *(tables[t].T for t in range(_CONV_SPLIT, N_CAT)))
    flats = tuple(flats_lo) + tuple(flats_hi)
    lin_tables = [f.reshape(_CAT_DIM, EMB_DIM) for f in flats]
    pieces = [_gather_kernels[t](lin_tables[t], idxs) for t in range(N_CAT)]
    out_t = _concat_kernel(cont, *pieces)
    return out_t.T


# speedup vs baseline: 1.3601x; 1.0931x over previous
import functools

import jax
import jax.numpy as jnp
from jax import lax
from jax.experimental import pallas as pl
from jax.experimental.pallas import tpu as pltpu
from jax.experimental.pallas import tpu_sc as plsc

BATCH = 16384
INPUT_DIM = 39
N_CONT = 13
N_CAT = 26
EMB_DIM = 16
OUT_DIM = N_CONT + N_CAT * EMB_DIM

_NUM_CORES = 2
_NUM_SUBCORES = 16
NW = _NUM_CORES * _NUM_SUBCORES
BPW = BATCH // NW
CHUNK = 128
N_CHUNKS = BPW // CHUNK
LANES = 16

_MESH = plsc.VectorSubcoreMesh(core_axis_name="c", subcore_axis_name="s")
_CPARAMS = pltpu.CompilerParams(use_tc_tiling_on_sc=False,
                                needs_layout_passes=False)


def _wid():
    return lax.axis_index("s") * _NUM_CORES + lax.axis_index("c")


def _extract_body(x_hbm, idxs_hbm, cont_hbm, x_v, i_stage, c_stage):
    base = _wid() * BPW
    iota = lax.iota(jnp.int32, LANES)
    pltpu.sync_copy(x_hbm.at[pl.ds(base, BPW)], x_v)

    def extract(b0, carry):
        rows = iota + b0 * LANES
        for j in range(INPUT_DIM):
            col = jnp.full((LANES,), j, jnp.int32)
            v = plsc.load_gather(x_v, [rows, col])
            if j < N_CONT:
                plsc.store_scatter(
                    c_stage, [jnp.full((LANES,), j, jnp.int32), rows],
                    v.astype(jnp.float32))
            else:
                plsc.store_scatter(
                    i_stage, [jnp.full((LANES,), j - N_CONT, jnp.int32),
                              rows], v)
        return carry

    lax.fori_loop(0, BPW // LANES, extract, 0)
    pltpu.sync_copy(i_stage, idxs_hbm.at[:, pl.ds(base, BPW)])
    pltpu.sync_copy(c_stage, cont_hbm.at[:, pl.ds(base, BPW)])


_extract_kernel = functools.partial(
    pl.kernel,
    out_type=(jax.ShapeDtypeStruct((N_CAT, BATCH), jnp.int32),
              jax.ShapeDtypeStruct((N_CONT, BATCH), jnp.float32)),
    mesh=_MESH,
    scratch_types=[
        pltpu.VMEM((BPW, INPUT_DIM), jnp.int32),
        pltpu.VMEM((N_CAT, BPW), jnp.int32),
        pltpu.VMEM((N_CONT, BPW), jnp.float32),
    ],
    compiler_params=_CPARAMS,
)(_extract_body)


_CPARAMS_TILED = pltpu.CompilerParams(use_tc_tiling_on_sc=True,
                                      needs_layout_passes=False)

_CAT_DIM = 100000
_COLS_PW = 3200
_CHUNKS = (1664, 1536)
_CHUNK_MAX = max(_CHUNKS)
_TAIL_W = NW - 1
_TAIL_MAIN = 768
_TAIL_CHUNKS = (384, 384)
_TAIL_PART = 32
_CONV_GROUPS = ((0, 7), (7, 13), (13, 20), (20, 26))


def _convert_body(tab_lo, tab_hi, *args):
    n = tab_hi - tab_lo
    tabs = args[:n]
    outs = args[n:2 * n]
    (slab_a, slab_b, stg_a, stg_b, tslab_a, tslab_b, tail_v, tstg_v,
     sem_in, sem_out) = args[2 * n:]

    wid = _wid()
    iota = lax.iota(jnp.int32, LANES)
    col0 = wid * _COLS_PW
    is_tail = wid == _TAIL_W

    def run_pipe(slabs, stgs, widths):
        steps = [(i, h) for i in range(n) for h in range(len(widths))]
        offs = [sum(widths[:h]) for h in range(len(widths))]

        def load(k):
            i, h = steps[k]
            w = widths[h]
            return pltpu.make_async_copy(
                tabs[i].at[:, pl.ds(col0 + offs[h], w)],
                slabs[k % 2].at[:, pl.ds(0, w)], sem_in)

        def store(k):
            i, h = steps[k]
            w = widths[h]
            return pltpu.make_async_copy(
                stgs[k % 2].at[pl.ds(0, w * EMB_DIM)],
                outs[i].at[pl.ds((col0 + offs[h]) * EMB_DIM,
                                 w * EMB_DIM)], sem_out)

        load(0).start()
        for k in range(len(steps)):
            i, h = steps[k]
            if k + 1 < len(steps):
                load(k + 1).start()
            load(k).wait()
            if k >= 2:
                store(k - 2).wait()
            stg = stgs[k % 2]
            slab = slabs[k % 2]

            def transpose(b0, carry):
                rcols = iota + b0 * LANES
                for e in range(EMB_DIM):
                    v = plsc.load_gather(
                        slab, [jnp.full((LANES,), e, jnp.int32), rcols])
                    plsc.store_scatter(stg, [rcols * EMB_DIM + e], v)
                return carry

            lax.fori_loop(0, widths[h] // LANES, transpose, 0)
            store(k).start()
        store(len(steps) - 2).wait()
        store(len(steps) - 1).wait()

    @pl.when(jnp.logical_not(is_tail))
    def _main():
        run_pipe((slab_a, slab_b), (stg_a, stg_b), _CHUNKS)

    @pl.when(is_tail)
    def _tail():
        run_pipe((tslab_a, tslab_b), (stg_a, stg_b), _TAIL_CHUNKS)
        for i in range(n):
            pltpu.sync_copy(
                tabs[i].at[:, pl.ds(_CAT_DIM - _TAIL_PART, _TAIL_PART)],
                tail_v)

            def ttail(b0, carry):
                rloc = iota + b0 * LANES
                for e in range(EMB_DIM):
                    v = plsc.load_gather(
                        tail_v, [jnp.full((LANES,), e, jnp.int32), rloc])
                    plsc.store_scatter(tstg_v, [rloc * EMB_DIM + e], v)
                return carry

            lax.fori_loop(0, _TAIL_PART // LANES, ttail, 0)
            pltpu.sync_copy(
                tstg_v,
                outs[i].at[pl.ds((_CAT_DIM - _TAIL_PART) * EMB_DIM,
                                 _TAIL_PART * EMB_DIM)])


def _make_convert_kernel(lo, hi):
    return functools.partial(
        pl.kernel,
        out_type=tuple(
            jax.ShapeDtypeStruct((_CAT_DIM * EMB_DIM,), jnp.float32)
            for _ in range(hi - lo)),
        mesh=_MESH,
        scratch_types=[
            pltpu.VMEM((EMB_DIM, _CHUNK_MAX), jnp.float32),
            pltpu.VMEM((EMB_DIM, _CHUNK_MAX), jnp.float32),
            pltpu.VMEM((_CHUNK_MAX * EMB_DIM,), jnp.float32),
            pltpu.VMEM((_CHUNK_MAX * EMB_DIM,), jnp.float32),
            pltpu.VMEM((EMB_DIM, _TAIL_CHUNKS[0]), jnp.float32),
            pltpu.VMEM((EMB_DIM, _TAIL_CHUNKS[0]), jnp.float32),
            pltpu.VMEM((EMB_DIM, _TAIL_PART), jnp.float32),
            pltpu.VMEM((_TAIL_PART * EMB_DIM,), jnp.float32),
            pltpu.SemaphoreType.DMA,
            pltpu.SemaphoreType.DMA,
        ],
        compiler_params=_CPARAMS_TILED,
        name=f"convert_{lo}_{hi}",
    )(functools.partial(_convert_body, lo, hi))


_convert_kernels = tuple(_make_convert_kernel(lo, hi)
                         for lo, hi in _CONV_GROUPS)


def _gather_body(t, table_hbm, idxs_hbm, piece_hbm, idx_v, ebuf, stage, sem):
    base = _wid() * BPW
    iota = lax.iota(jnp.int32, LANES)
    pltpu.sync_copy(idxs_hbm.at[t, pl.ds(base, BPW)], idx_v)
    gather = pltpu.make_async_copy(table_hbm.at[idx_v], ebuf, sem)
    gather.start()
    gather.wait()

    def asm(b0, carry):
        brows = iota + b0 * LANES
        off = pl.multiple_of(b0 * LANES, LANES)
        for e in range(EMB_DIM):
            v = plsc.load_gather(
                ebuf, [brows, jnp.full((LANES,), e, jnp.int32)])
            stage[e, pl.ds(off, LANES)] = v
        return carry

    lax.fori_loop(0, BPW // LANES, asm, 0)
    pltpu.sync_copy(stage, piece_hbm.at[:, pl.ds(base, BPW)])


def _make_gather_kernel(t):
    return functools.partial(
        pl.kernel,
        out_type=jax.ShapeDtypeStruct((EMB_DIM, BATCH), jnp.float32),
        mesh=_MESH,
        scratch_types=[
            pltpu.VMEM((BPW,), jnp.int32),
            pltpu.VMEM((BPW, EMB_DIM), jnp.float32),
            pltpu.VMEM((EMB_DIM, BPW), jnp.float32),
            pltpu.SemaphoreType.DMA,
        ],
        compiler_params=_CPARAMS,
        name=f"gather_t{t}",
    )(functools.partial(_gather_body, t))


_gather_kernels = [_make_gather_kernel(t) for t in range(N_CAT)]


def _concat_body(*args):
    cont_hbm = args[0]
    pieces = args[1:1 + N_CAT]
    out_hbm = args[1 + N_CAT]
    stage_v = args[2 + N_CAT]
    base = _wid() * BPW

    def chunk_body(c, carry):
        cb = base + c * CHUNK
        pltpu.sync_copy(cont_hbm.at[:, pl.ds(cb, CHUNK)],
                        stage_v.at[pl.ds(0, N_CONT)])
        for t in range(N_CAT):
            pltpu.sync_copy(
                pieces[t].at[:, pl.ds(cb, CHUNK)],
                stage_v.at[pl.ds(N_CONT + t * EMB_DIM, EMB_DIM)])
        pltpu.sync_copy(stage_v, out_hbm.at[:, pl.ds(cb, CHUNK)])
        return carry

    lax.fori_loop(0, N_CHUNKS, chunk_body, 0)


_concat_kernel = functools.partial(
    pl.kernel,
    out_type=jax.ShapeDtypeStruct((OUT_DIM, BATCH), jnp.float32),
    mesh=_MESH,
    scratch_types=[
        pltpu.VMEM((OUT_DIM, CHUNK), jnp.float32),
    ],
    compiler_params=_CPARAMS,
)(_concat_body)


def kernel(x, emb_0, emb_1, emb_2, emb_3, emb_4, emb_5, emb_6, emb_7,
           emb_8, emb_9, emb_10, emb_11, emb_12, emb_13, emb_14, emb_15,
           emb_16, emb_17, emb_18, emb_19, emb_20, emb_21, emb_22, emb_23,
           emb_24, emb_25):
    tables = (emb_0, emb_1, emb_2, emb_3, emb_4, emb_5, emb_6, emb_7,
              emb_8, emb_9, emb_10, emb_11, emb_12, emb_13, emb_14, emb_15,
              emb_16, emb_17, emb_18, emb_19, emb_20, emb_21, emb_22, emb_23,
              emb_24, emb_25)
    idxs, cont = _extract_kernel(x.astype(jnp.int32))
    flats = []
    for g, (lo, hi) in enumerate(_CONV_GROUPS):
        flats.extend(_convert_kernels[g](
            *(tables[t].T for t in range(lo, hi))))
    lin_tables = [f.reshape(_CAT_DIM, EMB_DIM) for f in flats]
    pieces = [_gather_kernels[t](lin_tables[t], idxs) for t in range(N_CAT)]
    out_t = _concat_kernel(cont, *pieces)
    return out_t.T


# speedup vs baseline: 1.5386x; 1.1312x over previous
import functools

import jax
import jax.numpy as jnp
from jax import lax
from jax.experimental import pallas as pl
from jax.experimental.pallas import tpu as pltpu
from jax.experimental.pallas import tpu_sc as plsc

BATCH = 16384
INPUT_DIM = 39
N_CONT = 13
N_CAT = 26
EMB_DIM = 16
OUT_DIM = N_CONT + N_CAT * EMB_DIM

_NUM_CORES = 2
_NUM_SUBCORES = 16
NW = _NUM_CORES * _NUM_SUBCORES
BPW = BATCH // NW
CHUNK = 128
N_CHUNKS = BPW // CHUNK
LANES = 16

_MESH = plsc.VectorSubcoreMesh(core_axis_name="c", subcore_axis_name="s")
_CPARAMS = pltpu.CompilerParams(use_tc_tiling_on_sc=False,
                                needs_layout_passes=False)


def _wid():
    return lax.axis_index("s") * _NUM_CORES + lax.axis_index("c")


def _extract_body(x_hbm, idxs_hbm, cont_hbm, x_v, i_stage, c_stage):
    base = _wid() * BPW
    iota = lax.iota(jnp.int32, LANES)
    pltpu.sync_copy(x_hbm.at[pl.ds(base, BPW)], x_v)

    def extract(b0, carry):
        rows = iota + b0 * LANES
        for j in range(INPUT_DIM):
            col = jnp.full((LANES,), j, jnp.int32)
            v = plsc.load_gather(x_v, [rows, col])
            if j < N_CONT:
                plsc.store_scatter(
                    c_stage, [jnp.full((LANES,), j, jnp.int32), rows],
                    v.astype(jnp.float32))
            else:
                plsc.store_scatter(
                    i_stage, [jnp.full((LANES,), j - N_CONT, jnp.int32),
                              rows], v)
        return carry

    lax.fori_loop(0, BPW // LANES, extract, 0)
    pltpu.sync_copy(i_stage, idxs_hbm.at[:, pl.ds(base, BPW)])
    pltpu.sync_copy(c_stage, cont_hbm.at[:, pl.ds(base, BPW)])


_extract_kernel = functools.partial(
    pl.kernel,
    out_type=(jax.ShapeDtypeStruct((N_CAT, BATCH), jnp.int32),
              jax.ShapeDtypeStruct((N_CONT, BATCH), jnp.float32)),
    mesh=_MESH,
    scratch_types=[
        pltpu.VMEM((BPW, INPUT_DIM), jnp.int32),
        pltpu.VMEM((N_CAT, BPW), jnp.int32),
        pltpu.VMEM((N_CONT, BPW), jnp.float32),
    ],
    compiler_params=_CPARAMS,
)(_extract_body)


_CPARAMS_TILED = pltpu.CompilerParams(use_tc_tiling_on_sc=True,
                                      needs_layout_passes=False)

_CAT_DIM = 100000
_COLS_PW = 3200
_CHUNKS = (1664, 1536)
_CHUNK_MAX = max(_CHUNKS)
_TAIL_W = NW - 1
_TAIL_MAIN = 768
_TAIL_CHUNKS = (384, 384)
_TAIL_PART = 32
_CONV_GROUPS = ((0, 7), (7, 13), (13, 20), (20, 26))


def _convert_body(tab_lo, tab_hi, *args):
    n = tab_hi - tab_lo
    tabs = args[:n]
    outs = args[n:2 * n]
    (slab_a, slab_b, stg_a, stg_b, tslab_a, tslab_b, tail_v, tstg_v,
     sem_in, sem_out) = args[2 * n:]

    wid = _wid()
    iota = lax.iota(jnp.int32, LANES)
    col0 = wid * _COLS_PW
    is_tail = wid == _TAIL_W

    def run_pipe(slabs, stgs, widths):
        steps = [(i, h) for i in range(n) for h in range(len(widths))]
        offs = [sum(widths[:h]) for h in range(len(widths))]

        def load(k):
            i, h = steps[k]
            w = widths[h]
            return pltpu.make_async_copy(
                tabs[i].at[:, pl.ds(col0 + offs[h], w)],
                slabs[k % 2].at[:, pl.ds(0, w)], sem_in)

        def store(k):
            i, h = steps[k]
            w = widths[h]
            return pltpu.make_async_copy(
                stgs[k % 2].at[pl.ds(0, w * EMB_DIM)],
                outs[i].at[pl.ds((col0 + offs[h]) * EMB_DIM,
                                 w * EMB_DIM)], sem_out)

        load(0).start()
        for k in range(len(steps)):
            i, h = steps[k]
            if k + 1 < len(steps):
                load(k + 1).start()
            load(k).wait()
            if k >= 2:
                store(k - 2).wait()
            stg = stgs[k % 2]
            slab = slabs[k % 2]

            def transpose(b0, carry):
                rcols = iota + b0 * LANES
                for e in range(EMB_DIM):
                    v = plsc.load_gather(
                        slab, [jnp.full((LANES,), e, jnp.int32), rcols])
                    plsc.store_scatter(stg, [rcols * EMB_DIM + e], v)
                return carry

            lax.fori_loop(0, widths[h] // LANES, transpose, 0)
            store(k).start()
        store(len(steps) - 2).wait()
        store(len(steps) - 1).wait()

    @pl.when(jnp.logical_not(is_tail))
    def _main():
        run_pipe((slab_a, slab_b), (stg_a, stg_b), _CHUNKS)

    @pl.when(is_tail)
    def _tail():
        run_pipe((tslab_a, tslab_b), (stg_a, stg_b), _TAIL_CHUNKS)
        for i in range(n):
            pltpu.sync_copy(
                tabs[i].at[:, pl.ds(_CAT_DIM - _TAIL_PART, _TAIL_PART)],
                tail_v)

            def ttail(b0, carry):
                rloc = iota + b0 * LANES
                for e in range(EMB_DIM):
                    v = plsc.load_gather(
                        tail_v, [jnp.full((LANES,), e, jnp.int32), rloc])
                    plsc.store_scatter(tstg_v, [rloc * EMB_DIM + e], v)
                return carry

            lax.fori_loop(0, _TAIL_PART // LANES, ttail, 0)
            pltpu.sync_copy(
                tstg_v,
                outs[i].at[pl.ds((_CAT_DIM - _TAIL_PART) * EMB_DIM,
                                 _TAIL_PART * EMB_DIM)])


def _make_convert_kernel(lo, hi):
    return functools.partial(
        pl.kernel,
        out_type=tuple(
            jax.ShapeDtypeStruct((_CAT_DIM * EMB_DIM,), jnp.float32)
            for _ in range(hi - lo)),
        mesh=_MESH,
        scratch_types=[
            pltpu.VMEM((EMB_DIM, _CHUNK_MAX), jnp.float32),
            pltpu.VMEM((EMB_DIM, _CHUNK_MAX), jnp.float32),
            pltpu.VMEM((_CHUNK_MAX * EMB_DIM,), jnp.float32),
            pltpu.VMEM((_CHUNK_MAX * EMB_DIM,), jnp.float32),
            pltpu.VMEM((EMB_DIM, _TAIL_CHUNKS[0]), jnp.float32),
            pltpu.VMEM((EMB_DIM, _TAIL_CHUNKS[0]), jnp.float32),
            pltpu.VMEM((EMB_DIM, _TAIL_PART), jnp.float32),
            pltpu.VMEM((_TAIL_PART * EMB_DIM,), jnp.float32),
            pltpu.SemaphoreType.DMA,
            pltpu.SemaphoreType.DMA,
        ],
        compiler_params=_CPARAMS_TILED,
        name=f"convert_{lo}_{hi}",
    )(functools.partial(_convert_body, lo, hi))


_convert_kernels = tuple(_make_convert_kernel(lo, hi)
                         for lo, hi in _CONV_GROUPS)


def _gather_body(lo, hi, *args):
    n = hi - lo
    tabs = args[:n]
    idxs_hbm = args[n]
    pieces = args[n + 1:2 * n + 1]
    rest = args[2 * n + 1:]
    idx_vs = rest[:n]
    ebufs = rest[n:2 * n]
    stage = rest[2 * n]
    sems = rest[2 * n + 1:]

    base = _wid() * BPW
    iota = lax.iota(jnp.int32, LANES)
    for j in range(n):
        pltpu.sync_copy(idxs_hbm.at[lo + j, pl.ds(base, BPW)], idx_vs[j])
        pltpu.make_async_copy(tabs[j].at[idx_vs[j]], ebufs[j],
                              sems[j]).start()

    for j in range(n):
        pltpu.make_async_copy(tabs[j].at[idx_vs[j]], ebufs[j],
                              sems[j]).wait()

        def asm(b0, carry, j=j):
            brows = iota + b0 * LANES
            off = pl.multiple_of(b0 * LANES, LANES)
            for e in range(EMB_DIM):
                v = plsc.load_gather(
                    ebufs[j], [brows, jnp.full((LANES,), e, jnp.int32)])
                stage[e, pl.ds(off, LANES)] = v
            return carry

        lax.fori_loop(0, BPW // LANES, asm, 0)
        pltpu.sync_copy(stage, pieces[j].at[:, pl.ds(base, BPW)])


_GATHER_GROUPS = ((0, 4), (4, 8), (8, 12), (12, 16), (16, 20), (20, 24),
                  (24, 26))


def _make_gather_kernel(lo, hi):
    n = hi - lo
    return functools.partial(
        pl.kernel,
        out_type=tuple(jax.ShapeDtypeStruct((EMB_DIM, BATCH), jnp.float32)
                       for _ in range(n)),
        mesh=_MESH,
        scratch_types=(
            [pltpu.VMEM((BPW,), jnp.int32) for _ in range(n)]
            + [pltpu.VMEM((BPW, EMB_DIM), jnp.float32) for _ in range(n)]
            + [pltpu.VMEM((EMB_DIM, BPW), jnp.float32)]
            + [pltpu.SemaphoreType.DMA for _ in range(n)]),
        compiler_params=_CPARAMS,
        name=f"gather_{lo}_{hi}",
    )(functools.partial(_gather_body, lo, hi))


_gather_kernels = [_make_gather_kernel(lo, hi) for lo, hi in _GATHER_GROUPS]


def _concat_body(*args):
    cont_hbm = args[0]
    pieces = args[1:1 + N_CAT]
    out_hbm = args[1 + N_CAT]
    stage_v = args[2 + N_CAT]
    base = _wid() * BPW

    def chunk_body(c, carry):
        cb = base + c * CHUNK
        pltpu.sync_copy(cont_hbm.at[:, pl.ds(cb, CHUNK)],
                        stage_v.at[pl.ds(0, N_CONT)])
        for t in range(N_CAT):
            pltpu.sync_copy(
                pieces[t].at[:, pl.ds(cb, CHUNK)],
                stage_v.at[pl.ds(N_CONT + t * EMB_DIM, EMB_DIM)])
        pltpu.sync_copy(stage_v, out_hbm.at[:, pl.ds(cb, CHUNK)])
        return carry

    lax.fori_loop(0, N_CHUNKS, chunk_body, 0)


_concat_kernel = functools.partial(
    pl.kernel,
    out_type=jax.ShapeDtypeStruct((OUT_DIM, BATCH), jnp.float32),
    mesh=_MESH,
    scratch_types=[
        pltpu.VMEM((OUT_DIM, CHUNK), jnp.float32),
    ],
    compiler_params=_CPARAMS,
)(_concat_body)


def kernel(x, emb_0, emb_1, emb_2, emb_3, emb_4, emb_5, emb_6, emb_7,
           emb_8, emb_9, emb_10, emb_11, emb_12, emb_13, emb_14, emb_15,
           emb_16, emb_17, emb_18, emb_19, emb_20, emb_21, emb_22, emb_23,
           emb_24, emb_25):
    tables = (emb_0, emb_1, emb_2, emb_3, emb_4, emb_5, emb_6, emb_7,
              emb_8, emb_9, emb_10, emb_11, emb_12, emb_13, emb_14, emb_15,
              emb_16, emb_17, emb_18, emb_19, emb_20, emb_21, emb_22, emb_23,
              emb_24, emb_25)
    idxs, cont = _extract_kernel(x.astype(jnp.int32))
    flats = []
    for g, (lo, hi) in enumerate(_CONV_GROUPS):
        flats.extend(_convert_kernels[g](
            *(tables[t].T for t in range(lo, hi))))
    lin_tables = [f.reshape(_CAT_DIM, EMB_DIM) for f in flats]
    pieces = []
    for g, (lo, hi) in enumerate(_GATHER_GROUPS):
        pieces.extend(_gather_kernels[g](
            *(lin_tables[t] for t in range(lo, hi)), idxs))
    out_t = _concat_kernel(cont, *pieces)
    return out_t.T
